# baseline (speedup 1.0000x reference)
import jax
import jax.numpy as jnp
from jax.experimental import pallas as pl
from jax.experimental.pallas import tpu as pltpu

T, N = 16, 1024
E, Hd, A = 64, 128, 32
Ps, Ph = 16, 16
V = 64
INF = 1e8
TN = 256


def _flat_product(rx, P, dtype):
    rx = rx.astype(dtype)
    tiled = jnp.concatenate([rx] * A, axis=-1)
    blk = jax.lax.broadcasted_iota(jnp.int32, (TN, A * E), 1) // E
    return jnp.where(blk == P, tiled, jnp.zeros((), dtype))


def _body(obs_ref, embed1_ref, WihT_ref, bih_ref, bhh_ref, We2T_ref, be2_ref,
          Wsh_ref, bsh_ref, WcrT_ref, bcr_ref, W1b_ref, W1b16_ref, bm1_ref,
          Wm2T_ref, Wm2T16_ref, bm2_ref, sl_out, X_out, vals_out, ml_out,
          el_out):
    f32 = jnp.float32
    obs = obs_ref[...]
    embed1 = embed1_ref[...]
    iotaV = jax.lax.broadcasted_iota(jnp.int32, (TN, V), 1)
    emb = []
    for t in range(T):
        oh_t = (obs[:, t:t + 1] == iotaV).astype(f32)
        emb.append(jnp.dot(oh_t, embed1, preferred_element_type=f32))

    bhh = bhh_ref[...]
    gi = jnp.dot(emb[0], WihT_ref[...], preferred_element_type=f32) + bih_ref[...]
    r = jax.nn.sigmoid(gi[:, :Hd] + bhh[:, :Hd])
    z = jax.nn.sigmoid(gi[:, Hd:2 * Hd] + bhh[:, Hd:2 * Hd])
    n = jnp.tanh(gi[:, 2 * Hd:] + r * bhh[:, 2 * Hd:])
    h1 = (1.0 - z) * n
    x0 = jnp.dot(h1, We2T_ref[...], preferred_element_type=f32) + be2_ref[...]

    iotaA = jax.lax.broadcasted_iota(jnp.int32, (TN, A), 1)
    Wsh = Wsh_ref[...]
    bsh = bsh_ref[...]
    W1b = W1b_ref[...]
    W1b16 = W1b16_ref[...]
    WcrT = WcrT_ref[...]
    Wm2T = Wm2T_ref[...]
    Wm2T16 = Wm2T16_ref[...]
    bcr = bcr_ref[...]
    bm1 = bm1_ref[...]
    bm2 = bm2_ref[...]
    zeroE = jnp.zeros((TN, E), f32)

    X = [x0] + [zeroE] * (Ph - 1)
    I = jnp.zeros((TN, 1), jnp.int32)
    logits = jnp.zeros((TN, A), f32)
    sl_list = []
    ohP = None
    P = None
    for _ in range(Ps):
        x = zeroE
        for p in range(Ph):
            x = x + jnp.where(I == p, X[p], 0.0)
        rx = jnp.maximum(x, 0.0)
        sharp = (rx * Wsh).sum(axis=-1, keepdims=True) + bsh
        values = jnp.dot(rx, WcrT, preferred_element_type=f32) + bcr
        is_new = jnp.all(logits == 0.0, axis=-1, keepdims=True)
        sl = jnp.where(is_new, sharp * values, logits)
        sl_list.append(sl)
        P = jnp.argmax(sl, axis=-1).astype(jnp.int32)[:, None]
        ohP = (iotaA == P).astype(f32)
        logits = sl - INF * ohP
        v_sel = (values * ohP).sum(axis=-1, keepdims=True)
        push = v_sel > 0.0
        flat = _flat_product(rx, P, f32)
        m1 = jnp.maximum(jnp.dot(flat, W1b, preferred_element_type=f32) + bm1, 0.0)
        m = jnp.dot(m1, Wm2T, preferred_element_type=f32) + bm2
        I_up = jnp.minimum(I + 1, Ph - 1)
        for p in range(Ph):
            X[p] = jnp.where(push & (I_up == p), m, X[p])
        I = jnp.where(push, I_up, jnp.maximum(I - 1, 0))

    sl_out[...] = jnp.concatenate(sl_list, axis=-1)
    X_out[...] = jnp.concatenate(X, axis=-1)
    vals_cols = [(sl_list[t] * ohP).sum(axis=-1, keepdims=True) for t in range(T)]
    vals_out[...] = jnp.concatenate(vals_cols, axis=-1)

    bf16 = jnp.bfloat16
    ml_cols = []
    for t in range(T):
        rp = jnp.maximum(emb[(t - 1) % T], 0.0)
        flat = _flat_product(rp, P, bf16)
        m1 = jnp.maximum(jnp.dot(flat, W1b16, preferred_element_type=f32)
                         + bm1, 0.0)
        m = jnp.dot(m1.astype(bf16), Wm2T16, preferred_element_type=f32) + bm2
        d = m - emb[t]
        ml_cols.append((d * d).mean(axis=-1, keepdims=True))
    ml_out[...] = jnp.concatenate(ml_cols, axis=-1)

    Xnorm = [jnp.sqrt((X[p] * X[p]).sum(axis=-1, keepdims=True)) for p in range(Ph)]
    el_cols = []
    for t in range(T):
        xt = emb[t]
        xtn = jnp.sqrt((xt * xt).sum(axis=-1, keepdims=True))
        sharp_t = (jnp.maximum(xt, 0.0) * Wsh).sum(axis=-1, keepdims=True) + bsh
        cos_cols = []
        for p in range(Ph):
            num = (X[p] * xt).sum(axis=-1, keepdims=True)
            cos_cols.append(num / (xtn * Xnorm[p] + 1e-8))
        s = sharp_t * jnp.concatenate(cos_cols, axis=-1)
        smax = jnp.max(s, axis=-1, keepdims=True)
        ex = jnp.exp(s - smax)
        lse = smax + jnp.log(ex.sum(axis=-1, keepdims=True))
        lp = s - lse
        el_cols.append((jnp.exp(lp) * lp).sum(axis=-1, keepdims=True))
    el_out[...] = jnp.concatenate(el_cols, axis=-1)


def kernel(obs, actions, rnn_hxs, embed1, Wih, Whh, bih, bhh, We2, be2, Wsh,
           bsh, Wcr, bcr, Wm1, bm1, Wm2, bm2):
    del actions, rnn_hxs, Whh
    obs2 = obs[:, :, 0].astype(jnp.int32).T
    W1b = Wm1.reshape(Hd, E, A).transpose(2, 1, 0).reshape(A * E, Hd)
    full = lambda a: pl.BlockSpec(a.shape, lambda i: (0,) * a.ndim)
    args = [
        embed1, Wih.T, bih[None], bhh[None], We2.T, be2[None], Wsh,
        bsh[None], Wcr.T, bcr[None], W1b, W1b.astype(jnp.bfloat16),
        bm1[None], Wm2.T, Wm2.T.astype(jnp.bfloat16), bm2[None],
    ]
    grid = (N // TN,)
    outs = pl.pallas_call(
        _body,
        grid=grid,
        in_specs=[pl.BlockSpec((TN, T), lambda i: (i, 0))] + [full(a) for a in args],
        out_specs=[
            pl.BlockSpec((TN, Ps * A), lambda i: (i, 0)),
            pl.BlockSpec((TN, Ph * E), lambda i: (i, 0)),
            pl.BlockSpec((TN, T), lambda i: (i, 0)),
            pl.BlockSpec((TN, T), lambda i: (i, 0)),
            pl.BlockSpec((TN, T), lambda i: (i, 0)),
        ],
        out_shape=[
            jax.ShapeDtypeStruct((N, Ps * A), jnp.float32),
            jax.ShapeDtypeStruct((N, Ph * E), jnp.float32),
            jax.ShapeDtypeStruct((N, T), jnp.float32),
            jax.ShapeDtypeStruct((N, T), jnp.float32),
            jax.ShapeDtypeStruct((N, T), jnp.float32),
        ],
        compiler_params=pltpu.CompilerParams(
            dimension_semantics=("arbitrary",)),
    )(obs2, *args)
    return jnp.concatenate(outs, axis=-1)


# speedup vs baseline: 1.8339x; 1.8339x over previous
import jax
import jax.numpy as jnp
from jax.experimental import pallas as pl
from jax.experimental.pallas import tpu as pltpu

T, N = 16, 1024
E, Hd, A = 64, 128, 32
Ps, Ph = 16, 16
V = 64
INF = 1e8
TN = 256
f32 = jnp.float32
bf16 = jnp.bfloat16


def _flat_product(rx16, P):
    tiled = jnp.concatenate([rx16] * A, axis=-1)
    blk = jax.lax.broadcasted_iota(jnp.int32, (TN, A * E), 1) // E
    return jnp.where(blk == P, tiled, jnp.zeros((), bf16))


def _body(obs_ref, embed1_ref, WihT_ref, bih_ref, bhh_ref, We2T_ref, be2_ref,
          Wsh_ref, bsh_ref, WcrT_ref, bcr_ref, W1b16_ref, bm1_ref,
          Wm2T16_ref, bm2_ref, S_ref, sl_out, X_out, vals_out, ml_out,
          el_out):
    obs = obs_ref[...]
    embed1 = embed1_ref[...]
    iotaV = jax.lax.broadcasted_iota(jnp.int32, (TN, V), 1)
    emb = []
    for t in range(T):
        oh_t = (obs[:, t:t + 1] == iotaV).astype(f32)
        emb.append(jnp.dot(oh_t, embed1, preferred_element_type=f32))

    bhh = bhh_ref[...]
    gi = jnp.dot(emb[0], WihT_ref[...], preferred_element_type=f32) + bih_ref[...]
    r = jax.nn.sigmoid(gi[:, :Hd] + bhh[:, :Hd])
    z = jax.nn.sigmoid(gi[:, Hd:2 * Hd] + bhh[:, Hd:2 * Hd])
    n = jnp.tanh(gi[:, 2 * Hd:] + r * bhh[:, 2 * Hd:])
    h1 = (1.0 - z) * n
    x0 = jnp.dot(h1, We2T_ref[...], preferred_element_type=f32) + be2_ref[...]

    iotaA = jax.lax.broadcasted_iota(jnp.int32, (TN, A), 1)
    Wsh = Wsh_ref[...]
    bsh = bsh_ref[...]
    W1b16 = W1b16_ref[...]
    WcrT = WcrT_ref[...]
    WcrT16 = WcrT.astype(bf16)
    Wm2T16 = Wm2T16_ref[...]
    bcr = bcr_ref[...]
    bm1 = bm1_ref[...]
    bm2 = bm2_ref[...]

    rx0 = jnp.maximum(x0, 0.0)
    sharp0 = (rx0 * Wsh).sum(axis=-1, keepdims=True) + bsh
    values0 = jnp.dot(rx0, WcrT, preferred_element_type=f32) + bcr
    sl0 = sharp0 * values0
    sl_list = [sl0]
    logits = sl0

    X = [x0.astype(bf16)] + [None] * (Ph - 1)
    zeroE16 = jnp.zeros((TN, E), bf16)
    I = jnp.zeros((TN, 1), jnp.int32)
    P = None
    for i in range(Ps):
        if i == 0:
            rx16 = rx0.astype(bf16)
            values = values0
        else:
            x = zeroE16
            for p in range(min(i, Ph - 1) + 1):
                x = x + jnp.where(I == p, X[p] if X[p] is not None else zeroE16, zeroE16)
            rx16 = jnp.maximum(x, zeroE16)
            values = jnp.dot(rx16, WcrT16, preferred_element_type=f32) + bcr
            sl_list.append(logits)
        P = jnp.argmax(logits, axis=-1).astype(jnp.int32)[:, None]
        ohP = (iotaA == P).astype(f32)
        logits = logits - INF * ohP
        v_sel = (values * ohP).sum(axis=-1, keepdims=True)
        push = v_sel > 0.0
        flat = _flat_product(rx16, P)
        m1 = jnp.maximum(jnp.dot(flat, W1b16, preferred_element_type=f32)
                         + bm1, 0.0)
        m = (jnp.dot(m1.astype(bf16), Wm2T16, preferred_element_type=f32)
             + bm2).astype(bf16)
        I_up = jnp.minimum(I + 1, Ph - 1)
        for p in range(1, min(i + 1, Ph - 1) + 1):
            old = X[p] if X[p] is not None else zeroE16
            X[p] = jnp.where(push & (I_up == p), m, old)
        I = jnp.where(push, I_up, jnp.maximum(I - 1, 0))
    X = [x if x is not None else zeroE16 for x in X]

    sl_out[...] = jnp.concatenate(sl_list, axis=-1)
    X2 = jnp.concatenate(X, axis=-1)
    X_out[...] = X2.astype(f32)
    ohPf = (iotaA == P).astype(f32)
    vals_cols = [(sl_list[t] * ohPf).sum(axis=-1, keepdims=True) for t in range(T)]
    vals_out[...] = jnp.concatenate(vals_cols, axis=-1)

    ml_cols = []
    for t in range(T):
        rp16 = jnp.maximum(emb[(t - 1) % T], 0.0).astype(bf16)
        flat = _flat_product(rp16, P)
        m1 = jnp.maximum(jnp.dot(flat, W1b16, preferred_element_type=f32)
                         + bm1, 0.0)
        m = jnp.dot(m1.astype(bf16), Wm2T16, preferred_element_type=f32) + bm2
        d = m - emb[t]
        ml_cols.append((d * d).mean(axis=-1, keepdims=True))
    ml_out[...] = jnp.concatenate(ml_cols, axis=-1)

    S16 = S_ref[...]
    EMBcat = jnp.concatenate(emb, axis=-1)
    EMB16 = EMBcat.astype(bf16)
    REMB16 = jnp.maximum(EMB16, jnp.zeros((), bf16))
    Wsh_t16 = jnp.concatenate([Wsh.astype(bf16)] * T, axis=-1)
    sharp_all = jnp.dot(REMB16 * Wsh_t16, S16,
                        preferred_element_type=f32) + bsh
    xtn2_all = jnp.dot(EMB16 * EMB16, S16, preferred_element_type=f32)
    Xn2 = jnp.dot(X2 * X2, S16, preferred_element_type=f32)
    Xnorm = jnp.sqrt(Xn2)
    el_cols = []
    for t in range(T):
        xt16 = emb[t].astype(bf16)
        xt_tiled = jnp.concatenate([xt16] * Ph, axis=-1)
        num = jnp.dot(X2 * xt_tiled, S16, preferred_element_type=f32)
        xtn = jnp.sqrt(xtn2_all[:, t:t + 1])
        cos = num / (xtn * Xnorm + 1e-8)
        s = sharp_all[:, t:t + 1] * cos
        smax = jnp.max(s, axis=-1, keepdims=True)
        ex = jnp.exp(s - smax)
        lse = smax + jnp.log(ex.sum(axis=-1, keepdims=True))
        lp = s - lse
        el_cols.append((jnp.exp(lp) * lp).sum(axis=-1, keepdims=True))
    el_out[...] = jnp.concatenate(el_cols, axis=-1)


def kernel(obs, actions, rnn_hxs, embed1, Wih, Whh, bih, bhh, We2, be2, Wsh,
           bsh, Wcr, bcr, Wm1, bm1, Wm2, bm2):
    del actions, rnn_hxs, Whh
    obs2 = obs[:, :, 0].astype(jnp.int32).T
    W1b = Wm1.reshape(Hd, E, A).transpose(2, 1, 0).reshape(A * E, Hd)
    S = jnp.repeat(jnp.eye(Ph, dtype=jnp.bfloat16), E, axis=0)
    full = lambda a: pl.BlockSpec(a.shape, lambda i: (0,) * a.ndim)
    args = [
        embed1, Wih.T, bih[None], bhh[None], We2.T, be2[None], Wsh,
        bsh[None], Wcr.T, bcr[None], W1b.astype(jnp.bfloat16),
        bm1[None], Wm2.T.astype(jnp.bfloat16), bm2[None], S,
    ]
    grid = (N // TN,)
    outs = pl.pallas_call(
        _body,
        grid=grid,
        in_specs=[pl.BlockSpec((TN, T), lambda i: (i, 0))] + [full(a) for a in args],
        out_specs=[
            pl.BlockSpec((TN, Ps * A), lambda i: (i, 0)),
            pl.BlockSpec((TN, Ph * E), lambda i: (i, 0)),
            pl.BlockSpec((TN, T), lambda i: (i, 0)),
            pl.BlockSpec((TN, T), lambda i: (i, 0)),
            pl.BlockSpec((TN, T), lambda i: (i, 0)),
        ],
        out_shape=[
            jax.ShapeDtypeStruct((N, Ps * A), jnp.float32),
            jax.ShapeDtypeStruct((N, Ph * E), jnp.float32),
            jax.ShapeDtypeStruct((N, T), jnp.float32),
            jax.ShapeDtypeStruct((N, T), jnp.float32),
            jax.ShapeDtypeStruct((N, T), jnp.float32),
        ],
        compiler_params=pltpu.CompilerParams(
            dimension_semantics=("arbitrary",)),
    )(obs2, *args)
    return jnp.concatenate(outs, axis=-1)
